# Initial kernel scaffold; baseline (speedup 1.0000x reference)
import jax
import jax.numpy as jnp
from jax.experimental import pallas as pl


def kernel(e_rec, s_rec, r_rec, n_rec, e_lig, s_lig, r_lig, n_lig, e_int, s_int, r_int, action, params):
    raise NotImplementedError("write your pallas kernel here")


# speedup vs baseline: 2.8526x; 2.8526x over previous
import functools

import jax
import jax.numpy as jnp
from jax import lax
from jax.experimental import pallas as pl
from jax.experimental.pallas import tpu as pltpu
from jax.experimental.pallas import tpu_sc as plsc

C = 128
F32 = jnp.float32
NC, NS = 2, 16
NW = NC * NS
N5, N10, N20 = 5000, 10000, 20000
E1 = 160000
ED = 480000
NPAD5, NPAD10 = 5120, 10240


def _relu(x):
    return jnp.maximum(x, 0.0)


def _dot(a, b):
    return jnp.dot(a, b, preferred_element_type=F32)


def _prep_body(A2, a1w, a1b, a2w, a2b, encnw, encnb, encew, enceb,
               se1w, se1b, ie1w, ie1b, de1w, se2w, se2b, ie2w, ie2b,
               sn1w, sn1b, in1w, in1b,
               o_iden, o_act, o_encn3, o_bn3, o_ences, o_encei,
               o_crec, o_clig, o_cint, o_e2ds, o_b2ds, o_e2di, o_b2di,
               o_w10, o_cgrec, o_cglig, o_cgint):
    v = _dot(_relu(_dot(A2[...], a1w[...]) + a1b[...]), a2w[...]) + a2b[...]
    iden = v[0:1, :]
    act = v[1:2, :]
    o_iden[...] = iden
    o_act[...] = act
    Ws = se1w[C:2 * C, :]
    Wr = se1w[2 * C:3 * C, :]
    o_encn3[...] = jnp.concatenate(
        [encnw[...], _dot(encnw[...], Ws), _dot(encnw[...], Wr)], axis=1)
    o_bn3[...] = jnp.concatenate(
        [encnb[...], _dot(encnb[...], Ws), _dot(encnb[...], Wr)], axis=1)
    Wes = se1w[0:C, :]
    Wei = ie1w[0:C, :]
    o_ences[...] = _dot(encew[...], Wes)
    o_encei[...] = _dot(encew[...], Wei)
    bes = _dot(enceb[...], Wes)
    bei = _dot(enceb[...], Wei)
    o_crec[...] = bes + _dot(iden, se1w[3 * C:, :]) + se1b[...]
    o_clig[...] = bes + _dot(act, se1w[3 * C:, :]) + se1b[...]
    o_cint[...] = bei + _dot(act, ie1w[3 * C:, :]) + ie1b[...]
    Wd = de1w[0:C, :]
    o_e2ds[...] = jnp.concatenate([se2w[...], _dot(se2w[...], Wd)], axis=1)
    o_b2ds[...] = jnp.concatenate([se2b[...], _dot(se2b[...], Wd)], axis=1)
    o_e2di[...] = jnp.concatenate([ie2w[...], _dot(ie2w[...], Wd)], axis=1)
    o_b2di[...] = jnp.concatenate([ie2b[...], _dot(ie2b[...], Wd)], axis=1)
    o_w10[...] = jnp.concatenate(
        [ie1w[C:2 * C, :], ie1w[2 * C:3 * C, :],
         de1w[C:2 * C, :], de1w[2 * C:3 * C, :]], axis=1)
    o_cgrec[...] = _dot(iden, sn1w[2 * C:, :]) + sn1b[...]
    o_cglig[...] = _dot(act, sn1w[2 * C:, :]) + sn1b[...]
    o_cgint[...] = _dot(act, in1w[2 * C:, :]) + in1b[...]


def _prep(*args):
    s = lambda *sh: jax.ShapeDtypeStruct(sh, F32)
    outs = (s(1, C), s(1, C), s(C, 3 * C), s(1, 3 * C), s(16, C), s(16, C),
            s(1, C), s(1, C), s(1, C), s(C, 2 * C), s(1, 2 * C),
            s(C, 2 * C), s(1, 2 * C), s(C, 4 * C), s(1, C), s(1, C), s(1, C))
    return pl.pallas_call(_prep_body, out_shape=outs)(*args)


def _dock_consts_body(grec, glig, gint, de1w, de1b, dn1w, dn1b,
                      o_g, o_c, o_cg):
    g = grec[...] + glig[...] + gint[...]
    o_g[...] = g
    o_c[...] = _dot(g, de1w[3 * C:, :]) + de1b[...]
    o_cg[...] = _dot(g, dn1w[2 * C:, :]) + dn1b[...]


def _dock_consts(*args):
    s = lambda *sh: jax.ShapeDtypeStruct(sh, F32)
    return pl.pallas_call(_dock_consts_body,
                          out_shape=(s(1, C), s(1, C), s(1, C)))(*args)


def _mm_bias_body(x, w, b, o):
    o[...] = _dot(x[...], w[...]) + b[...]


def _encode_nodes(x10, w, b):
    return pl.pallas_call(
        _mm_bias_body,
        out_shape=jax.ShapeDtypeStruct((N10, 3 * C), F32))(x10, w, b)


def _mm_body(x, w, o):
    o[...] = _dot(x[...], w[...])


def _encode_edges(e_raw, w):
    E = e_raw.shape[0]
    T = 2000
    return pl.pallas_call(
        _mm_body,
        grid=(E // T,),
        in_specs=[pl.BlockSpec((T, 16), lambda i: (i, 0)),
                  pl.BlockSpec((16, C), lambda i: (0, 0))],
        out_specs=pl.BlockSpec((T, C), lambda i: (i, 0)),
        out_shape=jax.ShapeDtypeStruct((E, C), F32))(e_raw, w)


def _proj10(t10, w10):
    return pl.pallas_call(
        _mm_body,
        out_shape=jax.ShapeDtypeStruct((N10, 4 * C), F32))(t10, w10)


def _combine_body(ep, gs, gr, c, w, b, o_eu, o_de):
    h = _relu(ep[...] + gs[...] + gr[...] + c[...])
    o = _dot(h, w[...]) + b[...]
    o_eu[...] = o[:, 0:C]
    o_de[...] = o[:, C:2 * C]


def _combine_edges(ep, gs, gr, c, w, b):
    E = ep.shape[0]
    T = 2000
    bs = lambda: pl.BlockSpec((T, C), lambda i: (i, 0))
    return pl.pallas_call(
        _combine_body,
        grid=(E // T,),
        in_specs=[bs(), bs(), bs(),
                  pl.BlockSpec((1, C), lambda i: (0, 0)),
                  pl.BlockSpec((C, 2 * C), lambda i: (0, 0)),
                  pl.BlockSpec((1, 2 * C), lambda i: (0, 0))],
        out_specs=(bs(), bs()),
        out_shape=(jax.ShapeDtypeStruct((E, C), F32),
                   jax.ShapeDtypeStruct((E, C), F32)))(ep, gs, gr, c, w, b)


def _combine_dock_body(de, gs, gr, c, w, b, o_eu):
    h = _relu(de[...] + gs[...] + gr[...] + c[...])
    o_eu[...] = _dot(h, w[...]) + b[...]


def _combine_dock(de, gs_full, gr_full, c, w, b, seg):
    T = 2000
    off = seg * (E1 // T)
    bs = pl.BlockSpec((T, C), lambda i: (i, 0))
    bso = pl.BlockSpec((T, C), lambda i, o=off: (o + i, 0))
    return pl.pallas_call(
        _combine_dock_body,
        grid=(E1 // T,),
        in_specs=[bs, bso, bso,
                  pl.BlockSpec((1, C), lambda i: (0, 0)),
                  pl.BlockSpec((C, C), lambda i: (0, 0)),
                  pl.BlockSpec((1, C), lambda i: (0, 0))],
        out_specs=bs,
        out_shape=jax.ShapeDtypeStruct((E1, C), F32))(
            de, gs_full, gr_full, c, w, b)


def _make_node_update(N, NPAD, E):
    def body(nodes, S, n1w, n2w, n2b, cg, g, g1w, g1b, g2w, g2b,
             o_nu, o_gu):
        agg = S[0:N, :] + S[NPAD:NPAD + N, :]
        hn = _relu(_dot(nodes[...], n1w[0:C, :]) + _dot(agg, n1w[C:2 * C, :])
                   + cg[...])
        nu = _dot(hn, n2w[...]) + n2b[...]
        o_nu[...] = nu
        mean_nu = jnp.sum(nu, axis=0, keepdims=True) * (1.0 / N)
        mean_eu = jnp.sum(agg, axis=0, keepdims=True) * (1.0 / E)
        gin = jnp.concatenate([mean_nu, mean_eu, g[...]], axis=1)
        o_gu[...] = _dot(_relu(_dot(gin, g1w[...]) + g1b[...]), g2w[...]) \
            + g2b[...]

    def call(*args):
        return pl.pallas_call(
            body,
            out_shape=(jax.ShapeDtypeStruct((N, C), F32),
                       jax.ShapeDtypeStruct((1, C), F32)))(*args)
    return call


_node_update5 = _make_node_update(N5, NPAD5, E1)
_node_update10 = _make_node_update(N10, NPAD10, E1)


def _dock_final_body(nodes, P, dn1w, dn2w, dn2b, cg, g, g1w, g1b, g2w, g2b,
                     outw, outb, valw, valb, o_q, acc):
    i = pl.program_id(0)

    @pl.when(i == 0)
    def _():
        acc[...] = jnp.zeros_like(acc)

    TN = 5000
    ii = jnp.minimum(i, 1)
    a0 = P[pl.ds(ii * TN, TN), :] + P[pl.ds(NPAD10 + ii * TN, TN), :]
    agg = a0 * jnp.where(i < 2, 1.0, 0.0)
    hn = _relu(_dot(nodes[...], dn1w[0:C, :]) + _dot(agg, dn1w[C:2 * C, :])
               + cg[...])
    nu = _dot(hn, dn2w[...]) + dn2b[...]
    acc[0:1, :] += jnp.sum(nu, axis=0, keepdims=True)
    acc[1:2, :] += jnp.sum(agg, axis=0, keepdims=True)

    @pl.when(i == 3)
    def _():
        mean_nu = acc[0:1, :] * (1.0 / N20)
        mean_eu = acc[1:2, :] * (1.0 / ED)
        gin = jnp.concatenate([mean_nu, mean_eu, g[...]], axis=1)
        gd = _dot(_relu(_dot(gin, g1w[...]) + g1b[...]), g2w[...]) + g2b[...]
        q = _relu(_dot(gd, outw[...]) + outb[...])
        q = jax.nn.sigmoid(_dot(q, valw[...]) + valb[...]) * 200.0 - 100.0
        o_q[...] = q


def _dock_final(nodes20, P, *weights):
    TN = 5000
    full = lambda r, c: pl.BlockSpec((r, c), lambda i: (0, 0))
    return pl.pallas_call(
        _dock_final_body,
        grid=(4,),
        in_specs=[pl.BlockSpec((TN, C), lambda i: (i, 0)),
                  full(2 * NPAD10, C),
                  full(3 * C, C), full(C, C), full(1, C), full(1, C),
                  full(1, C), full(3 * C, C), full(1, C), full(C, C),
                  full(1, C), full(C, C), full(1, C), full(C, 1),
                  full(1, 1)],
        out_specs=pl.BlockSpec((1, 1), lambda i: (0, 0)),
        out_shape=jax.ShapeDtypeStruct((1, 1), F32),
        scratch_shapes=[pltpu.VMEM((8, C), F32)])(nodes20, P, *weights)


def _make_sc_gather(E):
    EW = E // NW
    NFULL = EW // 128
    REM = EW - NFULL * 128
    mesh = plsc.VectorSubcoreMesh(core_axis_name="c", subcore_axis_name="s",
                                  num_cores=NC, num_subcores=NS)
    scratch = [pltpu.VMEM((128,), jnp.int32), pltpu.VMEM((128,), jnp.int32),
               pltpu.VMEM((128, C), F32), pltpu.VMEM((128, C), F32),
               pltpu.SemaphoreType.DMA, pltpu.SemaphoreType.DMA]
    if REM:
        scratch += [pltpu.VMEM((REM,), jnp.int32),
                    pltpu.VMEM((REM,), jnp.int32),
                    pltpu.VMEM((REM, C), F32), pltpu.VMEM((REM, C), F32)]

    def kern(T1, T2, I1, I2, O1, O2, i1v, i2v, r1v, r2v, sem1, sem2,
             *rem_scr):
        wid = lax.axis_index("s") * NC + lax.axis_index("c")
        base = wid * EW

        def do(off, i1, i2, r1, r2):
            pltpu.sync_copy(I1.at[pl.ds(off, i1.shape[0])], i1)
            pltpu.sync_copy(I2.at[pl.ds(off, i2.shape[0])], i2)
            c1 = pltpu.async_copy(T1.at[i1], r1, sem1)
            c2 = pltpu.async_copy(T2.at[i2], r2, sem2)
            c1.wait()
            c2.wait()
            pltpu.sync_copy(r1, O1.at[pl.ds(off, r1.shape[0])])
            pltpu.sync_copy(r2, O2.at[pl.ds(off, r2.shape[0])])

        def loop(j, carry):
            do(base + j * 128, i1v, i2v, r1v, r2v)
            return carry

        lax.fori_loop(0, NFULL, loop, 0)
        if REM:
            i1r, i2r, r1r, r2r = rem_scr
            do(base + NFULL * 128, i1r, i2r, r1r, r2r)

    out = (jax.ShapeDtypeStruct((E, C), F32), jax.ShapeDtypeStruct((E, C), F32))
    return pl.kernel(kern, out_type=out, mesh=mesh, scratch_types=scratch)


_sc_gather_e1 = _make_sc_gather(E1)
_sc_gather_ed = _make_sc_gather(ED)


def _make_sc_scatter(nseg, NPAD):
    EW = E1 // NW
    NFULL = EW // 128
    REM = EW - NFULL * 128
    RPS = NPAD // NS
    mesh = plsc.VectorSubcoreMesh(core_axis_name="c", subcore_axis_name="s",
                                  num_cores=NC, num_subcores=NS)
    scratch = [pltpu.VMEM_SHARED((NPAD, C), F32),
               pltpu.VMEM((128,), jnp.int32), pltpu.VMEM((128, C), F32),
               pltpu.VMEM((REM,), jnp.int32), pltpu.VMEM((REM, C), F32)]

    def kern(*refs):
        zed = refs[0]
        vi = refs[1:1 + 2 * nseg]
        out = refs[1 + 2 * nseg]
        S, iv, vv, ivr, vvr = refs[2 + 2 * nseg:]
        cid = lax.axis_index("c")
        sid = lax.axis_index("s")
        wid = sid * NC + cid
        pltpu.sync_copy(zed.at[pl.ds(sid * RPS, RPS)],
                        S.at[pl.ds(sid * RPS, RPS)])
        plsc.subcore_barrier()
        base = wid * EW
        for k in range(nseg):
            V, I = vi[2 * k], vi[2 * k + 1]

            def loop(j, carry):
                off = base + j * 128
                pltpu.sync_copy(I.at[pl.ds(off, 128)], iv)
                pltpu.sync_copy(V.at[pl.ds(off, 128)], vv)
                pltpu.sync_copy(vv, S.at[iv], add=True)
                return carry

            lax.fori_loop(0, NFULL, loop, 0)
            off = base + NFULL * 128
            pltpu.sync_copy(I.at[pl.ds(off, REM)], ivr)
            pltpu.sync_copy(V.at[pl.ds(off, REM)], vvr)
            pltpu.sync_copy(vvr, S.at[ivr], add=True)
        plsc.subcore_barrier()
        pltpu.sync_copy(S.at[pl.ds(sid * RPS, RPS)],
                        out.at[pl.ds(cid * NPAD + sid * RPS, RPS)])

    out = jax.ShapeDtypeStruct((2 * NPAD, C), F32)
    return pl.kernel(kern, out_type=out, mesh=mesh, scratch_types=scratch)


_sc_scatter5 = _make_sc_scatter(1, NPAD5)
_sc_scatter10 = _make_sc_scatter(1, NPAD10)
_sc_scatter_dock = _make_sc_scatter(3, NPAD10)


def kernel(e_rec, s_rec, r_rec, n_rec, e_lig, s_lig, r_lig, n_lig,
           e_int, s_int, r_int, action, params):
    p = params
    row = lambda v: v.reshape(1, -1)
    A2 = jnp.zeros((8, 8), F32).at[0, 0].set(1.0).at[1].set(action)

    (iden, act, encn3, bn3, ences, encei, c_rec, c_lig, c_int,
     e2ds, b2ds, e2di, b2di, w10, cg_rec, cg_lig, cg_int) = _prep(
        A2, p["act1"][0], row(p["act1"][1]), p["act2"][0], row(p["act2"][1]),
        p["enc_n"][0], row(p["enc_n"][1]), p["enc_e"][0], row(p["enc_e"][1]),
        p["single_e1"][0], row(p["single_e1"][1]),
        p["inter_e1"][0], row(p["inter_e1"][1]),
        p["dock_e1"][0],
        p["single_e2"][0], row(p["single_e2"][1]),
        p["inter_e2"][0], row(p["inter_e2"][1]),
        p["single_n1"][0], row(p["single_n1"][1]),
        p["inter_n1"][0], row(p["inter_n1"][1]))

    x10 = jnp.concatenate([n_rec, n_lig], axis=0)
    y = _encode_nodes(x10, encn3, bn3)
    nr, ps_rec, pr_rec = y[:N5, :C], y[:N5, C:2 * C], y[:N5, 2 * C:]
    nl, ps_lig, pr_lig = y[N5:, :C], y[N5:, C:2 * C], y[N5:, 2 * C:]

    ep_rec = _encode_edges(e_rec, ences)
    ep_lig = _encode_edges(e_lig, ences)
    ep_int = _encode_edges(e_int, encei)

    zed5 = jnp.zeros((NPAD5, C), F32)
    zed10 = jnp.zeros((NPAD10, C), F32)

    gs, gr = _sc_gather_e1(ps_rec, pr_rec, s_rec, r_rec)
    eu_rec, de_rec = _combine_edges(ep_rec, gs, gr, c_rec, e2ds, b2ds)
    part = _sc_scatter5(zed5, eu_rec, r_rec)
    nu_rec, grec = _node_update5(
        nr, part, p["single_n1"][0], p["single_n2"][0], row(p["single_n2"][1]),
        cg_rec, iden, p["single_g1"][0], row(p["single_g1"][1]),
        p["single_g2"][0], row(p["single_g2"][1]))

    gs, gr = _sc_gather_e1(ps_lig, pr_lig, s_lig, r_lig)
    eu_lig, de_lig = _combine_edges(ep_lig, gs, gr, c_lig, e2ds, b2ds)
    part = _sc_scatter5(zed5, eu_lig, r_lig)
    nu_lig, glig = _node_update5(
        nl, part, p["single_n1"][0], p["single_n2"][0], row(p["single_n2"][1]),
        cg_lig, act, p["single_g1"][0], row(p["single_g1"][1]),
        p["single_g2"][0], row(p["single_g2"][1]))

    t10 = jnp.concatenate([nu_rec, nu_lig], axis=0)
    p10 = _proj10(t10, w10)
    gs, gr = _sc_gather_e1(p10[:, :C], p10[:, C:2 * C], s_int, r_int)
    eu_int, de_int = _combine_edges(ep_int, gs, gr, c_int, e2di, b2di)
    part = _sc_scatter10(zed10, eu_int, r_int)
    nu_int, gint = _node_update10(
        t10, part, p["inter_n1"][0], p["inter_n2"][0], row(p["inter_n2"][1]),
        cg_int, act, p["inter_g1"][0], row(p["inter_g1"][1]),
        p["inter_g2"][0], row(p["inter_g2"][1]))

    g_dock, c_dock, cg_dock = _dock_consts(
        grec, glig, gint, p["dock_e1"][0], row(p["dock_e1"][1]),
        p["dock_n1"][0], row(p["dock_n1"][1]))
    s_dock = jnp.concatenate([s_rec, s_lig + 400, s_int])
    r_lig4 = r_lig + 400
    r_dock = jnp.concatenate([r_rec, r_lig4, r_int])
    gsd, grd = _sc_gather_ed(p10[:, 2 * C:3 * C], p10[:, 3 * C:],
                             s_dock, r_dock)
    eud_rec = _combine_dock(de_rec, gsd, grd, c_dock,
                            p["dock_e2"][0], row(p["dock_e2"][1]), 0)
    eud_lig = _combine_dock(de_lig, gsd, grd, c_dock,
                            p["dock_e2"][0], row(p["dock_e2"][1]), 1)
    eud_int = _combine_dock(de_int, gsd, grd, c_dock,
                            p["dock_e2"][0], row(p["dock_e2"][1]), 2)
    pd = _sc_scatter_dock(zed10, eud_rec, r_rec, eud_lig, r_lig4,
                          eud_int, r_int)

    nodes20 = jnp.concatenate([t10, nu_int], axis=0)
    q = _dock_final(
        nodes20, pd, p["dock_n1"][0], p["dock_n2"][0], row(p["dock_n2"][1]),
        cg_dock, g_dock, p["dock_g1"][0], row(p["dock_g1"][1]),
        p["dock_g2"][0], row(p["dock_g2"][1]),
        p["out"][0], row(p["out"][1]), p["value"][0], row(p["value"][1]))
    return q.reshape(1)


# speedup vs baseline: 3.3354x; 1.1692x over previous
import functools

import jax
import jax.numpy as jnp
from jax import lax
from jax.experimental import pallas as pl
from jax.experimental.pallas import tpu as pltpu
from jax.experimental.pallas import tpu_sc as plsc

C = 128
F32 = jnp.float32
NC, NS = 2, 16
NW = NC * NS
N5, N10, N20 = 5000, 10000, 20000
E1 = 160000
ED = 480000
NPAD5, NPAD10 = 5120, 10240


def _relu(x):
    return jnp.maximum(x, 0.0)


def _dot(a, b):
    return jnp.dot(a, b, preferred_element_type=F32)


def _prep_body(A2, a1w, a1b, a2w, a2b, encnw, encnb, encew, enceb,
               se1w, se1b, ie1w, ie1b, de1w, se2w, se2b, ie2w, ie2b,
               sn1w, sn1b, in1w, in1b,
               o_iden, o_act, o_encn3, o_bn3, o_ences, o_encei,
               o_crec, o_clig, o_cint, o_e2ds, o_b2ds, o_e2di, o_b2di,
               o_w10, o_cgrec, o_cglig, o_cgint):
    v = _dot(_relu(_dot(A2[...], a1w[...]) + a1b[...]), a2w[...]) + a2b[...]
    iden = v[0:1, :]
    act = v[1:2, :]
    o_iden[...] = iden
    o_act[...] = act
    Ws = se1w[C:2 * C, :]
    Wr = se1w[2 * C:3 * C, :]
    o_encn3[...] = jnp.concatenate(
        [encnw[...], _dot(encnw[...], Ws), _dot(encnw[...], Wr)], axis=1)
    o_bn3[...] = jnp.concatenate(
        [encnb[...], _dot(encnb[...], Ws), _dot(encnb[...], Wr)], axis=1)
    Wes = se1w[0:C, :]
    Wei = ie1w[0:C, :]
    o_ences[...] = _dot(encew[...], Wes)
    o_encei[...] = _dot(encew[...], Wei)
    bes = _dot(enceb[...], Wes)
    bei = _dot(enceb[...], Wei)
    o_crec[...] = bes + _dot(iden, se1w[3 * C:, :]) + se1b[...]
    o_clig[...] = bes + _dot(act, se1w[3 * C:, :]) + se1b[...]
    o_cint[...] = bei + _dot(act, ie1w[3 * C:, :]) + ie1b[...]
    Wd = de1w[0:C, :]
    o_e2ds[...] = jnp.concatenate([se2w[...], _dot(se2w[...], Wd)], axis=1)
    o_b2ds[...] = jnp.concatenate([se2b[...], _dot(se2b[...], Wd)], axis=1)
    o_e2di[...] = jnp.concatenate([ie2w[...], _dot(ie2w[...], Wd)], axis=1)
    o_b2di[...] = jnp.concatenate([ie2b[...], _dot(ie2b[...], Wd)], axis=1)
    o_w10[...] = jnp.concatenate(
        [ie1w[C:2 * C, :], ie1w[2 * C:3 * C, :],
         de1w[C:2 * C, :], de1w[2 * C:3 * C, :]], axis=1)
    o_cgrec[...] = _dot(iden, sn1w[2 * C:, :]) + sn1b[...]
    o_cglig[...] = _dot(act, sn1w[2 * C:, :]) + sn1b[...]
    o_cgint[...] = _dot(act, in1w[2 * C:, :]) + in1b[...]


def _prep(*args):
    s = lambda *sh: jax.ShapeDtypeStruct(sh, F32)
    outs = (s(1, C), s(1, C), s(C, 3 * C), s(1, 3 * C), s(16, C), s(16, C),
            s(1, C), s(1, C), s(1, C), s(C, 2 * C), s(1, 2 * C),
            s(C, 2 * C), s(1, 2 * C), s(C, 4 * C), s(1, C), s(1, C), s(1, C))
    return pl.pallas_call(_prep_body, out_shape=outs)(*args)


def _dock_consts_body(grec, glig, gint, de1w, de1b, dn1w, dn1b,
                      o_g, o_c, o_cg):
    g = grec[...] + glig[...] + gint[...]
    o_g[...] = g
    o_c[...] = _dot(g, de1w[3 * C:, :]) + de1b[...]
    o_cg[...] = _dot(g, dn1w[2 * C:, :]) + dn1b[...]


def _dock_consts(*args):
    s = lambda *sh: jax.ShapeDtypeStruct(sh, F32)
    return pl.pallas_call(_dock_consts_body,
                          out_shape=(s(1, C), s(1, C), s(1, C)))(*args)


def _mm_bias_body(x, w, b, o):
    o[...] = _dot(x[...], w[...]) + b[...]


def _encode_nodes(x10, w, b):
    return pl.pallas_call(
        _mm_bias_body,
        out_shape=jax.ShapeDtypeStruct((N10, 3 * C), F32))(x10, w, b)


def _mm_body(x, w, o):
    o[...] = _dot(x[...], w[...])


def _encode_edges(e_raw, w):
    E = e_raw.shape[0]
    T = 2000
    return pl.pallas_call(
        _mm_body,
        grid=(E // T,),
        in_specs=[pl.BlockSpec((T, 16), lambda i: (i, 0)),
                  pl.BlockSpec((16, C), lambda i: (0, 0))],
        out_specs=pl.BlockSpec((T, C), lambda i: (i, 0)),
        out_shape=jax.ShapeDtypeStruct((E, C), F32))(e_raw, w)


def _proj10(t10, w10):
    return pl.pallas_call(
        _mm_body,
        out_shape=jax.ShapeDtypeStruct((N10, 4 * C), F32))(t10, w10)


def _combine_body(ep, gs, gr, c, w, b, o_eu, o_de):
    h = _relu(ep[...] + gs[...] + gr[...] + c[...])
    o = _dot(h, w[...]) + b[...]
    o_eu[...] = o[:, 0:C]
    o_de[...] = o[:, C:2 * C]


def _combine_edges(ep, gs, gr, c, w, b):
    E = ep.shape[0]
    T = 2000
    bs = lambda: pl.BlockSpec((T, C), lambda i: (i, 0))
    return pl.pallas_call(
        _combine_body,
        grid=(E // T,),
        in_specs=[bs(), bs(), bs(),
                  pl.BlockSpec((1, C), lambda i: (0, 0)),
                  pl.BlockSpec((C, 2 * C), lambda i: (0, 0)),
                  pl.BlockSpec((1, 2 * C), lambda i: (0, 0))],
        out_specs=(bs(), bs()),
        out_shape=(jax.ShapeDtypeStruct((E, C), F32),
                   jax.ShapeDtypeStruct((E, C), F32)))(ep, gs, gr, c, w, b)


def _combine_dock_body(de, gs, gr, c, w, b, o_eu):
    h = _relu(de[...] + gs[...] + gr[...] + c[...])
    o_eu[...] = _dot(h, w[...]) + b[...]


def _combine_dock(de, gs_full, gr_full, c, w, b, seg):
    T = 2000
    off = seg * (E1 // T)
    bs = pl.BlockSpec((T, C), lambda i: (i, 0))
    bso = pl.BlockSpec((T, C), lambda i, o=off: (o + i, 0))
    return pl.pallas_call(
        _combine_dock_body,
        grid=(E1 // T,),
        in_specs=[bs, bso, bso,
                  pl.BlockSpec((1, C), lambda i: (0, 0)),
                  pl.BlockSpec((C, C), lambda i: (0, 0)),
                  pl.BlockSpec((1, C), lambda i: (0, 0))],
        out_specs=bs,
        out_shape=jax.ShapeDtypeStruct((E1, C), F32))(
            de, gs_full, gr_full, c, w, b)


def _make_node_update(N, NPAD, E):
    def body(nodes, S, n1w, n2w, n2b, cg, g, g1w, g1b, g2w, g2b,
             o_nu, o_gu):
        agg = S[0:N, :] + S[NPAD:NPAD + N, :]
        hn = _relu(_dot(nodes[...], n1w[0:C, :]) + _dot(agg, n1w[C:2 * C, :])
                   + cg[...])
        nu = _dot(hn, n2w[...]) + n2b[...]
        o_nu[...] = nu
        mean_nu = jnp.sum(nu, axis=0, keepdims=True) * (1.0 / N)
        mean_eu = jnp.sum(agg, axis=0, keepdims=True) * (1.0 / E)
        gin = jnp.concatenate([mean_nu, mean_eu, g[...]], axis=1)
        o_gu[...] = _dot(_relu(_dot(gin, g1w[...]) + g1b[...]), g2w[...]) \
            + g2b[...]

    def call(*args):
        return pl.pallas_call(
            body,
            out_shape=(jax.ShapeDtypeStruct((N, C), F32),
                       jax.ShapeDtypeStruct((1, C), F32)))(*args)
    return call


_node_update5 = _make_node_update(N5, NPAD5, E1)
_node_update10 = _make_node_update(N10, NPAD10, E1)


def _dock_final_body(nodes, P, dn1w, dn2w, dn2b, cg, g, g1w, g1b, g2w, g2b,
                     outw, outb, valw, valb, o_q, acc):
    i = pl.program_id(0)

    @pl.when(i == 0)
    def _():
        acc[...] = jnp.zeros_like(acc)

    TN = 5000
    ii = jnp.minimum(i, 1)
    a0 = P[pl.ds(ii * TN, TN), :] + P[pl.ds(NPAD10 + ii * TN, TN), :]
    agg = a0 * jnp.where(i < 2, 1.0, 0.0)
    hn = _relu(_dot(nodes[...], dn1w[0:C, :]) + _dot(agg, dn1w[C:2 * C, :])
               + cg[...])
    nu = _dot(hn, dn2w[...]) + dn2b[...]
    acc[0:1, :] += jnp.sum(nu, axis=0, keepdims=True)
    acc[1:2, :] += jnp.sum(agg, axis=0, keepdims=True)

    @pl.when(i == 3)
    def _():
        mean_nu = acc[0:1, :] * (1.0 / N20)
        mean_eu = acc[1:2, :] * (1.0 / ED)
        gin = jnp.concatenate([mean_nu, mean_eu, g[...]], axis=1)
        gd = _dot(_relu(_dot(gin, g1w[...]) + g1b[...]), g2w[...]) + g2b[...]
        q = _relu(_dot(gd, outw[...]) + outb[...])
        q = jax.nn.sigmoid(_dot(q, valw[...]) + valb[...]) * 200.0 - 100.0
        o_q[...] = q


def _dock_final(nodes20, P, *weights):
    TN = 5000
    full = lambda r, c: pl.BlockSpec((r, c), lambda i: (0, 0))
    return pl.pallas_call(
        _dock_final_body,
        grid=(4,),
        in_specs=[pl.BlockSpec((TN, C), lambda i: (i, 0)),
                  full(2 * NPAD10, C),
                  full(3 * C, C), full(C, C), full(1, C), full(1, C),
                  full(1, C), full(3 * C, C), full(1, C), full(C, C),
                  full(1, C), full(C, C), full(1, C), full(C, 1),
                  full(1, 1)],
        out_specs=pl.BlockSpec((1, 1), lambda i: (0, 0)),
        out_shape=jax.ShapeDtypeStruct((1, 1), F32),
        scratch_shapes=[pltpu.VMEM((8, C), F32)])(nodes20, P, *weights)


def _make_sc_gather(E, K):
    EW = E // NW
    NFULL = EW // 128
    REM = EW - NFULL * 128
    NG = NFULL // K
    TAIL = NFULL - NG * K
    mesh = plsc.VectorSubcoreMesh(core_axis_name="c", subcore_axis_name="s",
                                  num_cores=NC, num_subcores=NS)
    scratch = [pltpu.VMEM((EW,), jnp.int32), pltpu.VMEM((EW,), jnp.int32),
               pltpu.VMEM((K, 128, C), F32), pltpu.VMEM((K, 128, C), F32),
               pltpu.SemaphoreType.DMA, pltpu.SemaphoreType.DMA]

    def kern(T1, T2, I1, I2, O1, O2, iv1, iv2, rb1, rb2, sem_g, sem_w):
        wid = lax.axis_index("s") * NC + lax.axis_index("c")
        base = wid * EW
        pltpu.sync_copy(I1.at[pl.ds(base, EW)], iv1)
        pltpu.sync_copy(I2.at[pl.ds(base, EW)], iv2)

        def group(goff, nk):
            gds = []
            for k in range(nk):
                loc = (goff + k) * 128
                gds.append(pltpu.async_copy(
                    T1.at[iv1.at[pl.ds(loc, 128)]], rb1.at[k], sem_g))
                gds.append(pltpu.async_copy(
                    T2.at[iv2.at[pl.ds(loc, 128)]], rb2.at[k], sem_g))
            wds = []
            for k in range(nk):
                gds[2 * k].wait()
                gds[2 * k + 1].wait()
                glob = base + (goff + k) * 128
                wds.append(pltpu.async_copy(
                    rb1.at[k], O1.at[pl.ds(glob, 128)], sem_w))
                wds.append(pltpu.async_copy(
                    rb2.at[k], O2.at[pl.ds(glob, 128)], sem_w))
            for d in wds:
                d.wait()

        def loop(g, carry):
            group(g * K, K)
            return carry

        lax.fori_loop(0, NG, loop, 0)
        if TAIL:
            group(NG * K, TAIL)
        if REM:
            loc = NFULL * 128
            d1 = pltpu.async_copy(
                T1.at[iv1.at[pl.ds(loc, REM)]],
                rb1.at[0, pl.ds(0, REM)], sem_g)
            d2 = pltpu.async_copy(
                T2.at[iv2.at[pl.ds(loc, REM)]],
                rb2.at[0, pl.ds(0, REM)], sem_g)
            d1.wait()
            d2.wait()
            w1 = pltpu.async_copy(
                rb1.at[0, pl.ds(0, REM)], O1.at[pl.ds(base + loc, REM)],
                sem_w)
            w2 = pltpu.async_copy(
                rb2.at[0, pl.ds(0, REM)], O2.at[pl.ds(base + loc, REM)],
                sem_w)
            w1.wait()
            w2.wait()

    out = (jax.ShapeDtypeStruct((E, C), F32), jax.ShapeDtypeStruct((E, C), F32))
    return pl.kernel(kern, out_type=out, mesh=mesh, scratch_types=scratch)


_sc_gather_e1 = _make_sc_gather(E1, 3)
_sc_gather_ed = _make_sc_gather(ED, 2)


def _make_sc_scatter(nseg, NPAD, K):
    EW = E1 // NW
    NFULL = EW // 128
    REM = EW - NFULL * 128
    NGRP = NFULL // K
    TAILK = NFULL - NGRP * K
    RPS = NPAD // NS
    mesh = plsc.VectorSubcoreMesh(core_axis_name="c", subcore_axis_name="s",
                                  num_cores=NC, num_subcores=NS)
    scratch = ([pltpu.VMEM_SHARED((NPAD, C), F32),
                pltpu.VMEM((K, 128, C), F32)]
               + [pltpu.VMEM((128,), jnp.int32) for _ in range(K)]
               + [pltpu.VMEM((REM,), jnp.int32), pltpu.VMEM((REM, C), F32),
                  pltpu.SemaphoreType.DMA, pltpu.SemaphoreType.DMA])

    def kern(*refs):
        zed = refs[0]
        vi = refs[1:1 + 2 * nseg]
        out = refs[1 + 2 * nseg]
        rest = refs[2 + 2 * nseg:]
        S, vb = rest[0], rest[1]
        ibs = rest[2:2 + K]
        ibr, vbr, sem_v, sem_s = rest[2 + K:]
        cid = lax.axis_index("c")
        sid = lax.axis_index("s")
        wid = sid * NC + cid
        pltpu.sync_copy(zed.at[pl.ds(sid * RPS, RPS)],
                        S.at[pl.ds(sid * RPS, RPS)])
        plsc.subcore_barrier()
        base = wid * EW
        for seg in range(nseg):
            V, I = vi[2 * seg], vi[2 * seg + 1]

            def group(go, nk):
                lds = []
                for k in range(nk):
                    off = base + (go + k) * 128
                    lds.append(pltpu.async_copy(
                        I.at[pl.ds(off, 128)], ibs[k], sem_v))
                    lds.append(pltpu.async_copy(
                        V.at[pl.ds(off, 128)], vb.at[k], sem_v))
                sds = []
                for k in range(nk):
                    lds[2 * k].wait()
                    lds[2 * k + 1].wait()
                    sds.append(pltpu.async_copy(
                        vb.at[k], S.at[ibs[k]], sem_s, add=True))
                for d in sds:
                    d.wait()

            def loop(g, carry):
                group(g * K, K)
                return carry

            lax.fori_loop(0, NGRP, loop, 0)
            if TAILK:
                group(NGRP * K, TAILK)
            if REM:
                off = base + NFULL * 128
                pltpu.sync_copy(I.at[pl.ds(off, REM)], ibr)
                pltpu.sync_copy(V.at[pl.ds(off, REM)], vbr)
                pltpu.sync_copy(vbr, S.at[ibr], add=True)
        plsc.subcore_barrier()
        pltpu.sync_copy(S.at[pl.ds(sid * RPS, RPS)],
                        out.at[pl.ds(cid * NPAD + sid * RPS, RPS)])

    out = jax.ShapeDtypeStruct((2 * NPAD, C), F32)
    return pl.kernel(kern, out_type=out, mesh=mesh, scratch_types=scratch)


_sc_scatter5 = _make_sc_scatter(1, NPAD5, 4)
_sc_scatter10 = _make_sc_scatter(1, NPAD10, 2)
_sc_scatter_dock = _make_sc_scatter(3, NPAD10, 2)


def kernel(e_rec, s_rec, r_rec, n_rec, e_lig, s_lig, r_lig, n_lig,
           e_int, s_int, r_int, action, params):
    p = params
    row = lambda v: v.reshape(1, -1)
    A2 = jnp.zeros((8, 8), F32).at[0, 0].set(1.0).at[1].set(action)

    (iden, act, encn3, bn3, ences, encei, c_rec, c_lig, c_int,
     e2ds, b2ds, e2di, b2di, w10, cg_rec, cg_lig, cg_int) = _prep(
        A2, p["act1"][0], row(p["act1"][1]), p["act2"][0], row(p["act2"][1]),
        p["enc_n"][0], row(p["enc_n"][1]), p["enc_e"][0], row(p["enc_e"][1]),
        p["single_e1"][0], row(p["single_e1"][1]),
        p["inter_e1"][0], row(p["inter_e1"][1]),
        p["dock_e1"][0],
        p["single_e2"][0], row(p["single_e2"][1]),
        p["inter_e2"][0], row(p["inter_e2"][1]),
        p["single_n1"][0], row(p["single_n1"][1]),
        p["inter_n1"][0], row(p["inter_n1"][1]))

    x10 = jnp.concatenate([n_rec, n_lig], axis=0)
    y = _encode_nodes(x10, encn3, bn3)
    nr, ps_rec, pr_rec = y[:N5, :C], y[:N5, C:2 * C], y[:N5, 2 * C:]
    nl, ps_lig, pr_lig = y[N5:, :C], y[N5:, C:2 * C], y[N5:, 2 * C:]

    ep_rec = _encode_edges(e_rec, ences)
    ep_lig = _encode_edges(e_lig, ences)
    ep_int = _encode_edges(e_int, encei)

    zed5 = jnp.zeros((NPAD5, C), F32)
    zed10 = jnp.zeros((NPAD10, C), F32)

    gs, gr = _sc_gather_e1(ps_rec, pr_rec, s_rec, r_rec)
    eu_rec, de_rec = _combine_edges(ep_rec, gs, gr, c_rec, e2ds, b2ds)
    part = _sc_scatter5(zed5, eu_rec, r_rec)
    nu_rec, grec = _node_update5(
        nr, part, p["single_n1"][0], p["single_n2"][0], row(p["single_n2"][1]),
        cg_rec, iden, p["single_g1"][0], row(p["single_g1"][1]),
        p["single_g2"][0], row(p["single_g2"][1]))

    gs, gr = _sc_gather_e1(ps_lig, pr_lig, s_lig, r_lig)
    eu_lig, de_lig = _combine_edges(ep_lig, gs, gr, c_lig, e2ds, b2ds)
    part = _sc_scatter5(zed5, eu_lig, r_lig)
    nu_lig, glig = _node_update5(
        nl, part, p["single_n1"][0], p["single_n2"][0], row(p["single_n2"][1]),
        cg_lig, act, p["single_g1"][0], row(p["single_g1"][1]),
        p["single_g2"][0], row(p["single_g2"][1]))

    t10 = jnp.concatenate([nu_rec, nu_lig], axis=0)
    p10 = _proj10(t10, w10)
    gs, gr = _sc_gather_e1(p10[:, :C], p10[:, C:2 * C], s_int, r_int)
    eu_int, de_int = _combine_edges(ep_int, gs, gr, c_int, e2di, b2di)
    part = _sc_scatter10(zed10, eu_int, r_int)
    nu_int, gint = _node_update10(
        t10, part, p["inter_n1"][0], p["inter_n2"][0], row(p["inter_n2"][1]),
        cg_int, act, p["inter_g1"][0], row(p["inter_g1"][1]),
        p["inter_g2"][0], row(p["inter_g2"][1]))

    g_dock, c_dock, cg_dock = _dock_consts(
        grec, glig, gint, p["dock_e1"][0], row(p["dock_e1"][1]),
        p["dock_n1"][0], row(p["dock_n1"][1]))
    s_dock = jnp.concatenate([s_rec, s_lig + 400, s_int])
    r_lig4 = r_lig + 400
    r_dock = jnp.concatenate([r_rec, r_lig4, r_int])
    gsd, grd = _sc_gather_ed(p10[:, 2 * C:3 * C], p10[:, 3 * C:],
                             s_dock, r_dock)
    eud_rec = _combine_dock(de_rec, gsd, grd, c_dock,
                            p["dock_e2"][0], row(p["dock_e2"][1]), 0)
    eud_lig = _combine_dock(de_lig, gsd, grd, c_dock,
                            p["dock_e2"][0], row(p["dock_e2"][1]), 1)
    eud_int = _combine_dock(de_int, gsd, grd, c_dock,
                            p["dock_e2"][0], row(p["dock_e2"][1]), 2)
    pd = _sc_scatter_dock(zed10, eud_rec, r_rec, eud_lig, r_lig4,
                          eud_int, r_int)

    nodes20 = jnp.concatenate([t10, nu_int], axis=0)
    q = _dock_final(
        nodes20, pd, p["dock_n1"][0], p["dock_n2"][0], row(p["dock_n2"][1]),
        cg_dock, g_dock, p["dock_g1"][0], row(p["dock_g1"][1]),
        p["dock_g2"][0], row(p["dock_g2"][1]),
        p["out"][0], row(p["out"][1]), p["value"][0], row(p["value"][1]))
    return q.reshape(1)


# speedup vs baseline: 3.4800x; 1.0434x over previous
import functools

import jax
import jax.numpy as jnp
from jax import lax
from jax.experimental import pallas as pl
from jax.experimental.pallas import tpu as pltpu
from jax.experimental.pallas import tpu_sc as plsc

C = 128
F32 = jnp.float32
NC, NS = 2, 16
NW = NC * NS
N5, N10, N20 = 5000, 10000, 20000
E1 = 160000
ED = 480000
NPAD5, NPAD10 = 5120, 10240


def _relu(x):
    return jnp.maximum(x, 0.0)


def _dot(a, b):
    return jnp.dot(a, b, preferred_element_type=F32)


def _prep_body(A2, a1w, a1b, a2w, a2b, encnw, encnb, encew, enceb,
               se1w, se1b, ie1w, ie1b, de1w, se2w, se2b, ie2w, ie2b,
               sn1w, sn1b, in1w, in1b,
               o_iden, o_act, o_encn3, o_bn3, o_ences, o_encei,
               o_crec, o_clig, o_cint, o_e2ds, o_b2ds, o_e2di, o_b2di,
               o_w10, o_cgrec, o_cglig, o_cgint):
    v = _dot(_relu(_dot(A2[...], a1w[...]) + a1b[...]), a2w[...]) + a2b[...]
    iden = v[0:1, :]
    act = v[1:2, :]
    o_iden[...] = iden
    o_act[...] = act
    Ws = se1w[C:2 * C, :]
    Wr = se1w[2 * C:3 * C, :]
    o_encn3[...] = jnp.concatenate(
        [encnw[...], _dot(encnw[...], Ws), _dot(encnw[...], Wr)], axis=1)
    o_bn3[...] = jnp.concatenate(
        [encnb[...], _dot(encnb[...], Ws), _dot(encnb[...], Wr)], axis=1)
    Wes = se1w[0:C, :]
    Wei = ie1w[0:C, :]
    o_ences[...] = _dot(encew[...], Wes)
    o_encei[...] = _dot(encew[...], Wei)
    bes = _dot(enceb[...], Wes)
    bei = _dot(enceb[...], Wei)
    o_crec[...] = bes + _dot(iden, se1w[3 * C:, :]) + se1b[...]
    o_clig[...] = bes + _dot(act, se1w[3 * C:, :]) + se1b[...]
    o_cint[...] = bei + _dot(act, ie1w[3 * C:, :]) + ie1b[...]
    Wd = de1w[0:C, :]
    o_e2ds[...] = jnp.concatenate([se2w[...], _dot(se2w[...], Wd)], axis=1)
    o_b2ds[...] = jnp.concatenate([se2b[...], _dot(se2b[...], Wd)], axis=1)
    o_e2di[...] = jnp.concatenate([ie2w[...], _dot(ie2w[...], Wd)], axis=1)
    o_b2di[...] = jnp.concatenate([ie2b[...], _dot(ie2b[...], Wd)], axis=1)
    o_w10[...] = jnp.concatenate(
        [ie1w[C:2 * C, :], ie1w[2 * C:3 * C, :],
         de1w[C:2 * C, :], de1w[2 * C:3 * C, :]], axis=1)
    o_cgrec[...] = _dot(iden, sn1w[2 * C:, :]) + sn1b[...]
    o_cglig[...] = _dot(act, sn1w[2 * C:, :]) + sn1b[...]
    o_cgint[...] = _dot(act, in1w[2 * C:, :]) + in1b[...]


def _prep(*args):
    s = lambda *sh: jax.ShapeDtypeStruct(sh, F32)
    outs = (s(1, C), s(1, C), s(C, 3 * C), s(1, 3 * C), s(16, C), s(16, C),
            s(1, C), s(1, C), s(1, C), s(C, 2 * C), s(1, 2 * C),
            s(C, 2 * C), s(1, 2 * C), s(C, 4 * C), s(1, C), s(1, C), s(1, C))
    return pl.pallas_call(_prep_body, out_shape=outs)(*args)


def _dock_consts_body(grec, glig, gint, de1w, de1b, dn1w, dn1b,
                      o_g, o_c, o_cg):
    g = grec[...] + glig[...] + gint[...]
    o_g[...] = g
    o_c[...] = _dot(g, de1w[3 * C:, :]) + de1b[...]
    o_cg[...] = _dot(g, dn1w[2 * C:, :]) + dn1b[...]


def _dock_consts(*args):
    s = lambda *sh: jax.ShapeDtypeStruct(sh, F32)
    return pl.pallas_call(_dock_consts_body,
                          out_shape=(s(1, C), s(1, C), s(1, C)))(*args)


def _mm_bias_body(x, w, b, o):
    o[...] = _dot(x[...], w[...]) + b[...]


def _encode_nodes(x10, w, b):
    return pl.pallas_call(
        _mm_bias_body,
        out_shape=jax.ShapeDtypeStruct((N10, 3 * C), F32))(x10, w, b)


def _mm_body(x, w, o):
    o[...] = _dot(x[...], w[...])


def _encode_edges(e_raw, w):
    E = e_raw.shape[0]
    T = 2000
    return pl.pallas_call(
        _mm_body,
        grid=(E // T,),
        in_specs=[pl.BlockSpec((T, 16), lambda i: (i, 0)),
                  pl.BlockSpec((16, C), lambda i: (0, 0))],
        out_specs=pl.BlockSpec((T, C), lambda i: (i, 0)),
        out_shape=jax.ShapeDtypeStruct((E, C), F32))(e_raw, w)


def _proj10(t10, w10):
    return pl.pallas_call(
        _mm_body,
        out_shape=jax.ShapeDtypeStruct((N10, 4 * C), F32))(t10, w10)


def _de_body(h, w, b, o):
    o[...] = _dot(h[...], w[...]) + b[...]


def _de_proj(h, w, b):
    T = 2000
    bs = pl.BlockSpec((T, C), lambda i: (i, 0))
    return pl.pallas_call(
        _de_body,
        grid=(E1 // T,),
        in_specs=[bs, pl.BlockSpec((C, C), lambda i: (0, 0)),
                  pl.BlockSpec((1, C), lambda i: (0, 0))],
        out_specs=bs,
        out_shape=jax.ShapeDtypeStruct((E1, C), F32))(h, w, b)


def _combine_dock_body(de, g, c, w, b, o_eu):
    h = _relu(de[...] + g[...] + c[...])
    o_eu[...] = _dot(h, w[...]) + b[...]


def _combine_dock(de, g_full, c, w, b, seg):
    T = 2000
    off = seg * (E1 // T)
    bs = pl.BlockSpec((T, C), lambda i: (i, 0))
    bso = pl.BlockSpec((T, C), lambda i, o=off: (o + i, 0))
    return pl.pallas_call(
        _combine_dock_body,
        grid=(E1 // T,),
        in_specs=[bs, bso,
                  pl.BlockSpec((1, C), lambda i: (0, 0)),
                  pl.BlockSpec((C, C), lambda i: (0, 0)),
                  pl.BlockSpec((1, C), lambda i: (0, 0))],
        out_specs=bs,
        out_shape=jax.ShapeDtypeStruct((E1, C), F32))(
            de, g_full, c, w, b)


def _make_node_update(N, NPAD, E):
    def body(nodes, S, cnt, e2w, e2b, n1w, n2w, n2b, cg, g, g1w, g1b, g2w, g2b,
             o_nu, o_gu):
        ssum = S[0:N, :] + S[NPAD:NPAD + N, :]
        cn = cnt[0:N, :] + cnt[NPAD:NPAD + N, :]
        agg = _dot(ssum, e2w[...]) + cn * e2b[...]
        hn = _relu(_dot(nodes[...], n1w[0:C, :]) + _dot(agg, n1w[C:2 * C, :])
                   + cg[...])
        nu = _dot(hn, n2w[...]) + n2b[...]
        o_nu[...] = nu
        mean_nu = jnp.sum(nu, axis=0, keepdims=True) * (1.0 / N)
        mean_eu = jnp.sum(agg, axis=0, keepdims=True) * (1.0 / E)
        gin = jnp.concatenate([mean_nu, mean_eu, g[...]], axis=1)
        o_gu[...] = _dot(_relu(_dot(gin, g1w[...]) + g1b[...]), g2w[...]) \
            + g2b[...]

    def call(*args):
        return pl.pallas_call(
            body,
            out_shape=(jax.ShapeDtypeStruct((N, C), F32),
                       jax.ShapeDtypeStruct((1, C), F32)))(*args)
    return call


_node_update5 = _make_node_update(N5, NPAD5, E1)
_node_update10 = _make_node_update(N10, NPAD10, E1)


def _dock_final_body(nodes, P, dn1w, dn2w, dn2b, cg, g, g1w, g1b, g2w, g2b,
                     outw, outb, valw, valb, o_q, acc):
    i = pl.program_id(0)

    @pl.when(i == 0)
    def _():
        acc[...] = jnp.zeros_like(acc)

    TN = 5000
    ii = jnp.minimum(i, 1)
    a0 = P[pl.ds(ii * TN, TN), :] + P[pl.ds(NPAD10 + ii * TN, TN), :]
    agg = a0 * jnp.where(i < 2, 1.0, 0.0)
    hn = _relu(_dot(nodes[...], dn1w[0:C, :]) + _dot(agg, dn1w[C:2 * C, :])
               + cg[...])
    nu = _dot(hn, dn2w[...]) + dn2b[...]
    acc[0:1, :] += jnp.sum(nu, axis=0, keepdims=True)
    acc[1:2, :] += jnp.sum(agg, axis=0, keepdims=True)

    @pl.when(i == 3)
    def _():
        mean_nu = acc[0:1, :] * (1.0 / N20)
        mean_eu = acc[1:2, :] * (1.0 / ED)
        gin = jnp.concatenate([mean_nu, mean_eu, g[...]], axis=1)
        gd = _dot(_relu(_dot(gin, g1w[...]) + g1b[...]), g2w[...]) + g2b[...]
        q = _relu(_dot(gd, outw[...]) + outb[...])
        q = jax.nn.sigmoid(_dot(q, valw[...]) + valb[...]) * 200.0 - 100.0
        o_q[...] = q


def _dock_final(nodes20, P, *weights):
    TN = 5000
    full = lambda r, c: pl.BlockSpec((r, c), lambda i: (0, 0))
    return pl.pallas_call(
        _dock_final_body,
        grid=(4,),
        in_specs=[pl.BlockSpec((TN, C), lambda i: (i, 0)),
                  full(2 * NPAD10, C),
                  full(3 * C, C), full(C, C), full(1, C), full(1, C),
                  full(1, C), full(3 * C, C), full(1, C), full(C, C),
                  full(1, C), full(C, C), full(1, C), full(C, 1),
                  full(1, 1)],
        out_specs=pl.BlockSpec((1, 1), lambda i: (0, 0)),
        out_shape=jax.ShapeDtypeStruct((1, 1), F32),
        scratch_shapes=[pltpu.VMEM((8, C), F32)])(nodes20, P, *weights)


def _make_sc_gather(E, K):
    EW = E // NW
    NFULL = EW // 128
    REM = EW - NFULL * 128
    NG = NFULL // K
    TAIL = NFULL - NG * K
    mesh = plsc.VectorSubcoreMesh(core_axis_name="c", subcore_axis_name="s",
                                  num_cores=NC, num_subcores=NS)
    scratch = [pltpu.VMEM((EW,), jnp.int32), pltpu.VMEM((EW,), jnp.int32),
               pltpu.VMEM((K, 128, C), F32), pltpu.VMEM((K, 128, C), F32),
               pltpu.SemaphoreType.DMA, pltpu.SemaphoreType.DMA]

    def kern(T1, T2, I1, I2, O1, iv1, iv2, rb1, rb2, sem_g, sem_w):
        wid = lax.axis_index("s") * NC + lax.axis_index("c")
        base = wid * EW
        pltpu.sync_copy(I1.at[pl.ds(base, EW)], iv1)
        pltpu.sync_copy(I2.at[pl.ds(base, EW)], iv2)

        def vsum(dst, src, nrow):
            def row(j, carry):
                for g in range(C // 16):
                    dst[j, pl.ds(g * 16, 16)] = (
                        dst[j, pl.ds(g * 16, 16)] + src[j, pl.ds(g * 16, 16)])
                return carry
            lax.fori_loop(0, nrow, row, 0)

        def group(goff, nk):
            gds = []
            for k in range(nk):
                loc = (goff + k) * 128
                gds.append(pltpu.async_copy(
                    T1.at[iv1.at[pl.ds(loc, 128)]], rb1.at[k], sem_g))
                gds.append(pltpu.async_copy(
                    T2.at[iv2.at[pl.ds(loc, 128)]], rb2.at[k], sem_g))
            wds = []
            for k in range(nk):
                gds[2 * k].wait()
                gds[2 * k + 1].wait()
                vsum(rb1.at[k], rb2.at[k], 128)
                glob = base + (goff + k) * 128
                wds.append(pltpu.async_copy(
                    rb1.at[k], O1.at[pl.ds(glob, 128)], sem_w))
            for d in wds:
                d.wait()

        def loop(g, carry):
            group(g * K, K)
            return carry

        lax.fori_loop(0, NG, loop, 0)
        if TAIL:
            group(NG * K, TAIL)
        if REM:
            loc = NFULL * 128
            d1 = pltpu.async_copy(
                T1.at[iv1.at[pl.ds(loc, REM)]],
                rb1.at[0, pl.ds(0, REM)], sem_g)
            d2 = pltpu.async_copy(
                T2.at[iv2.at[pl.ds(loc, REM)]],
                rb2.at[0, pl.ds(0, REM)], sem_g)
            d1.wait()
            d2.wait()
            vsum(rb1.at[0], rb2.at[0], REM)
            w1 = pltpu.async_copy(
                rb1.at[0, pl.ds(0, REM)], O1.at[pl.ds(base + loc, REM)],
                sem_w)
            w1.wait()

    out = jax.ShapeDtypeStruct((E, C), F32)
    return pl.kernel(kern, out_type=out, mesh=mesh, scratch_types=scratch)


_sc_gather_ed = _make_sc_gather(ED, 2)


def _make_sc_scatter(nseg, NPAD, K):
    EW = E1 // NW
    NFULL = EW // 128
    REM = EW - NFULL * 128
    NGRP = NFULL // K
    TAILK = NFULL - NGRP * K
    RPS = NPAD // NS
    mesh = plsc.VectorSubcoreMesh(core_axis_name="c", subcore_axis_name="s",
                                  num_cores=NC, num_subcores=NS)
    scratch = ([pltpu.VMEM_SHARED((NPAD, C), F32),
                pltpu.VMEM((K, 128, C), F32)]
               + [pltpu.VMEM((128,), jnp.int32) for _ in range(K)]
               + [pltpu.VMEM((REM,), jnp.int32), pltpu.VMEM((REM, C), F32),
                  pltpu.SemaphoreType.DMA, pltpu.SemaphoreType.DMA])

    def kern(*refs):
        zed = refs[0]
        vi = refs[1:1 + 2 * nseg]
        out = refs[1 + 2 * nseg]
        rest = refs[2 + 2 * nseg:]
        S, vb = rest[0], rest[1]
        ibs = rest[2:2 + K]
        ibr, vbr, sem_v, sem_s = rest[2 + K:]
        cid = lax.axis_index("c")
        sid = lax.axis_index("s")
        wid = sid * NC + cid
        pltpu.sync_copy(zed.at[pl.ds(sid * RPS, RPS)],
                        S.at[pl.ds(sid * RPS, RPS)])
        plsc.subcore_barrier()
        base = wid * EW
        for seg in range(nseg):
            V, I = vi[2 * seg], vi[2 * seg + 1]

            def group(go, nk):
                lds = []
                for k in range(nk):
                    off = base + (go + k) * 128
                    lds.append(pltpu.async_copy(
                        I.at[pl.ds(off, 128)], ibs[k], sem_v))
                    lds.append(pltpu.async_copy(
                        V.at[pl.ds(off, 128)], vb.at[k], sem_v))
                sds = []
                for k in range(nk):
                    lds[2 * k].wait()
                    lds[2 * k + 1].wait()
                    sds.append(pltpu.async_copy(
                        vb.at[k], S.at[ibs[k]], sem_s, add=True))
                for d in sds:
                    d.wait()

            def loop(g, carry):
                group(g * K, K)
                return carry

            lax.fori_loop(0, NGRP, loop, 0)
            if TAILK:
                group(NGRP * K, TAILK)
            if REM:
                off = base + NFULL * 128
                pltpu.sync_copy(I.at[pl.ds(off, REM)], ibr)
                pltpu.sync_copy(V.at[pl.ds(off, REM)], vbr)
                pltpu.sync_copy(vbr, S.at[ibr], add=True)
        plsc.subcore_barrier()
        pltpu.sync_copy(S.at[pl.ds(sid * RPS, RPS)],
                        out.at[pl.ds(cid * NPAD + sid * RPS, RPS)])

    out = jax.ShapeDtypeStruct((2 * NPAD, C), F32)
    return pl.kernel(kern, out_type=out, mesh=mesh, scratch_types=scratch)


_sc_scatter_dock = _make_sc_scatter(3, NPAD10, 2)


def _make_sc_counts(npads):
    EW = E1 // NW
    NFULL = EW // 128
    REM = EW - NFULL * 128
    K = 4
    NG = NFULL // K
    TAILK = NFULL - NG * K
    nseg = len(npads)
    mesh = plsc.VectorSubcoreMesh(core_axis_name="c", subcore_axis_name="s",
                                  num_cores=NC, num_subcores=NS)
    scratch = ([pltpu.VMEM_SHARED((NPAD, C), F32) for NPAD in npads]
               + [pltpu.VMEM((128, C), F32)]
               + [pltpu.VMEM((128,), jnp.int32) for _ in range(K)]
               + [pltpu.VMEM((REM,), jnp.int32),
                  pltpu.SemaphoreType.DMA, pltpu.SemaphoreType.DMA])

    def kern(*refs):
        zeds = refs[:nseg]
        ones_in = refs[nseg]
        idxs = refs[nseg + 1:1 + 2 * nseg]
        outs = refs[1 + 2 * nseg:1 + 3 * nseg]
        rest = refs[1 + 3 * nseg:]
        CTs = rest[:nseg]
        ob = rest[nseg]
        ibs = rest[nseg + 1:nseg + 1 + K]
        ibr, sem_i, sem_s = rest[nseg + 1 + K:]
        cid = lax.axis_index("c")
        sid = lax.axis_index("s")
        wid = sid * NC + cid
        for NPAD, zed, CT in zip(npads, zeds, CTs):
            rps = NPAD // NS
            pltpu.sync_copy(zed.at[pl.ds(sid * rps, rps)],
                            CT.at[pl.ds(sid * rps, rps)])
        pltpu.sync_copy(ones_in, ob)
        plsc.subcore_barrier()
        base = wid * EW
        for I, CT in zip(idxs, CTs):
            def group(go, nk, I=I, CT=CT):
                lds = []
                for k in range(nk):
                    lds.append(pltpu.async_copy(
                        I.at[pl.ds(base + (go + k) * 128, 128)], ibs[k],
                        sem_i))
                sds = []
                for k in range(nk):
                    lds[k].wait()
                    sds.append(pltpu.async_copy(
                        ob, CT.at[ibs[k]], sem_s, add=True))
                for d in sds:
                    d.wait()

            def loop(g, carry, group=group):
                group(g * K, K)
                return carry

            lax.fori_loop(0, NG, loop, 0)
            if TAILK:
                group(NG * K, TAILK)
            if REM:
                pltpu.sync_copy(I.at[pl.ds(base + NFULL * 128, REM)], ibr)
                pltpu.sync_copy(ob.at[pl.ds(0, REM)], CT.at[ibr], add=True)
        plsc.subcore_barrier()
        for NPAD, CT, o in zip(npads, CTs, outs):
            rps = NPAD // NS
            pltpu.sync_copy(CT.at[pl.ds(sid * rps, rps)],
                            o.at[pl.ds(cid * NPAD + sid * rps, rps)])

    out = tuple(jax.ShapeDtypeStruct((2 * NPAD, C), F32) for NPAD in npads)
    if nseg == 1:
        out = out[0]
    return pl.kernel(kern, out_type=out, mesh=mesh, scratch_types=scratch)


_sc_counts_a = _make_sc_counts([NPAD5, NPAD5])
_sc_counts_b = _make_sc_counts([NPAD10])


def _make_sc_block(NPAD, CH, K):
    EW = E1 // NW
    NFULL = EW // CH
    REM = EW - NFULL * CH
    NG = NFULL // K
    TAILK = NFULL - NG * K
    RPS = NPAD // NS
    mesh = plsc.VectorSubcoreMesh(core_axis_name="c", subcore_axis_name="s",
                                  num_cores=NC, num_subcores=NS)
    scratch = ([pltpu.VMEM_SHARED((NPAD, C), F32),
                pltpu.VMEM((EW,), jnp.int32),
                pltpu.VMEM((K, CH, C), F32), pltpu.VMEM((K, CH, C), F32),
                pltpu.VMEM((K, CH, C), F32)]
               + [pltpu.VMEM((CH,), jnp.int32) for _ in range(K)]
               + [pltpu.VMEM((REM,), jnp.int32), pltpu.VMEM((1, C), F32)]
               + [pltpu.SemaphoreType.DMA for _ in range(5)])

    def kern(T1, T2, SI, RI, EP, cvec, zed, OH, OP, *rest):
        S, sv, epb, gsb, grb = rest[:5]
        ribs = rest[5:5 + K]
        ribr, cb, sem_i, sem_v, sem_g, sem_w, sem_s = rest[5 + K:]
        cid = lax.axis_index("c")
        sid = lax.axis_index("s")
        wid = sid * NC + cid
        base = wid * EW
        pltpu.sync_copy(zed.at[pl.ds(sid * RPS, RPS)],
                        S.at[pl.ds(sid * RPS, RPS)])
        pltpu.sync_copy(SI.at[pl.ds(base, EW)], sv)
        pltpu.sync_copy(cvec, cb)
        plsc.subcore_barrier()

        def fuse(dst, a2, a3, nrow):
            def rowf(j, cr):
                for g in range(C // 16):
                    sl = (j, pl.ds(g * 16, 16))
                    dst[sl] = jnp.maximum(
                        dst[sl] + a2[sl] + a3[sl] + cb[0, pl.ds(g * 16, 16)],
                        0.0)
                return cr
            lax.fori_loop(0, nrow, rowf, 0)

        def group(go, nk):
            drs, deps, dgs = [], [], []
            for k in range(nk):
                loc = (go + k) * CH
                glob = base + loc
                drs.append(pltpu.async_copy(
                    RI.at[pl.ds(glob, CH)], ribs[k], sem_i))
                deps.append(pltpu.async_copy(
                    EP.at[pl.ds(glob, CH)], epb.at[k], sem_v))
                dgs.append(pltpu.async_copy(
                    T1.at[sv.at[pl.ds(loc, CH)]], gsb.at[k], sem_g))
            dgr = []
            for k in range(nk):
                drs[k].wait()
                dgr.append(pltpu.async_copy(
                    T2.at[ribs[k]], grb.at[k], sem_g))
            fin = []
            for k in range(nk):
                deps[k].wait()
                dgs[k].wait()
                dgr[k].wait()
                fuse(gsb.at[k], grb.at[k], epb.at[k], CH)
                glob = base + (go + k) * CH
                fin.append(pltpu.async_copy(
                    gsb.at[k], OH.at[pl.ds(glob, CH)], sem_w))
                fin.append(pltpu.async_copy(
                    gsb.at[k], S.at[ribs[k]], sem_s, add=True))
            for d in fin:
                d.wait()

        def loop(g, carry):
            group(g * K, K)
            return carry

        lax.fori_loop(0, NG, loop, 0)
        if TAILK:
            group(NG * K, TAILK)
        if REM:
            loc = NFULL * CH
            glob = base + loc
            pltpu.sync_copy(RI.at[pl.ds(glob, REM)], ribr)
            dep = pltpu.async_copy(
                EP.at[pl.ds(glob, REM)], epb.at[0, pl.ds(0, REM)], sem_v)
            dg1 = pltpu.async_copy(
                T1.at[sv.at[pl.ds(loc, REM)]], gsb.at[0, pl.ds(0, REM)],
                sem_g)
            dg2 = pltpu.async_copy(
                T2.at[ribr], grb.at[0, pl.ds(0, REM)], sem_g)
            dep.wait()
            dg1.wait()
            dg2.wait()
            fuse(gsb.at[0], grb.at[0], epb.at[0], REM)
            w1 = pltpu.async_copy(
                gsb.at[0, pl.ds(0, REM)], OH.at[pl.ds(glob, REM)], sem_w)
            s1 = pltpu.async_copy(
                gsb.at[0, pl.ds(0, REM)], S.at[ribr], sem_s, add=True)
            w1.wait()
            s1.wait()
        plsc.subcore_barrier()
        pltpu.sync_copy(S.at[pl.ds(sid * RPS, RPS)],
                        OP.at[pl.ds(cid * NPAD + sid * RPS, RPS)])

    out = (jax.ShapeDtypeStruct((E1, C), F32),
           jax.ShapeDtypeStruct((2 * NPAD, C), F32))
    return pl.kernel(kern, out_type=out, mesh=mesh, scratch_types=scratch)


_sc_block5 = _make_sc_block(NPAD5, 64, 3)
_sc_block10 = _make_sc_block(NPAD10, 48, 2)


def kernel(e_rec, s_rec, r_rec, n_rec, e_lig, s_lig, r_lig, n_lig,
           e_int, s_int, r_int, action, params):
    p = params
    row = lambda v: v.reshape(1, -1)
    A2 = jnp.zeros((8, 8), F32).at[0, 0].set(1.0).at[1].set(action)

    (iden, act, encn3, bn3, ences, encei, c_rec, c_lig, c_int,
     e2ds, b2ds, e2di, b2di, w10, cg_rec, cg_lig, cg_int) = _prep(
        A2, p["act1"][0], row(p["act1"][1]), p["act2"][0], row(p["act2"][1]),
        p["enc_n"][0], row(p["enc_n"][1]), p["enc_e"][0], row(p["enc_e"][1]),
        p["single_e1"][0], row(p["single_e1"][1]),
        p["inter_e1"][0], row(p["inter_e1"][1]),
        p["dock_e1"][0],
        p["single_e2"][0], row(p["single_e2"][1]),
        p["inter_e2"][0], row(p["inter_e2"][1]),
        p["single_n1"][0], row(p["single_n1"][1]),
        p["inter_n1"][0], row(p["inter_n1"][1]))

    x10 = jnp.concatenate([n_rec, n_lig], axis=0)
    y = _encode_nodes(x10, encn3, bn3)
    nr, ps_rec, pr_rec = y[:N5, :C], y[:N5, C:2 * C], y[:N5, 2 * C:]
    nl, ps_lig, pr_lig = y[N5:, :C], y[N5:, C:2 * C], y[N5:, 2 * C:]

    ep_rec = _encode_edges(e_rec, ences)
    ep_lig = _encode_edges(e_lig, ences)
    ep_int = _encode_edges(e_int, encei)

    zed5 = jnp.zeros((NPAD5, C), F32)
    zed10 = jnp.zeros((NPAD10, C), F32)
    ones128 = jnp.ones((128, C), F32)
    cnt_rec, cnt_lig = _sc_counts_a(zed5, zed5, ones128, r_rec, r_lig)
    cnt_int = _sc_counts_b(zed10, ones128, r_int)

    h_rec, part = _sc_block5(ps_rec, pr_rec, s_rec, r_rec, ep_rec, c_rec,
                             zed5)
    de_rec = _de_proj(h_rec, e2ds[:, C:], b2ds[:, C:])
    nu_rec, grec = _node_update5(
        nr, part, cnt_rec, p["single_e2"][0], row(p["single_e2"][1]),
        p["single_n1"][0], p["single_n2"][0], row(p["single_n2"][1]),
        cg_rec, iden, p["single_g1"][0], row(p["single_g1"][1]),
        p["single_g2"][0], row(p["single_g2"][1]))

    h_lig, part = _sc_block5(ps_lig, pr_lig, s_lig, r_lig, ep_lig, c_lig,
                             zed5)
    de_lig = _de_proj(h_lig, e2ds[:, C:], b2ds[:, C:])
    nu_lig, glig = _node_update5(
        nl, part, cnt_lig, p["single_e2"][0], row(p["single_e2"][1]),
        p["single_n1"][0], p["single_n2"][0], row(p["single_n2"][1]),
        cg_lig, act, p["single_g1"][0], row(p["single_g1"][1]),
        p["single_g2"][0], row(p["single_g2"][1]))

    t10 = jnp.concatenate([nu_rec, nu_lig], axis=0)
    p10 = _proj10(t10, w10)
    h_int, part = _sc_block10(p10[:, :C], p10[:, C:2 * C], s_int, r_int,
                              ep_int, c_int, zed10)
    de_int = _de_proj(h_int, e2di[:, C:], b2di[:, C:])
    nu_int, gint = _node_update10(
        t10, part, cnt_int, p["inter_e2"][0], row(p["inter_e2"][1]),
        p["inter_n1"][0], p["inter_n2"][0], row(p["inter_n2"][1]),
        cg_int, act, p["inter_g1"][0], row(p["inter_g1"][1]),
        p["inter_g2"][0], row(p["inter_g2"][1]))

    g_dock, c_dock, cg_dock = _dock_consts(
        grec, glig, gint, p["dock_e1"][0], row(p["dock_e1"][1]),
        p["dock_n1"][0], row(p["dock_n1"][1]))
    s_dock = jnp.concatenate([s_rec, s_lig + 400, s_int])
    r_lig4 = r_lig + 400
    r_dock = jnp.concatenate([r_rec, r_lig4, r_int])
    gd_sum = _sc_gather_ed(p10[:, 2 * C:3 * C], p10[:, 3 * C:],
                           s_dock, r_dock)
    eud_rec = _combine_dock(de_rec, gd_sum, c_dock,
                            p["dock_e2"][0], row(p["dock_e2"][1]), 0)
    eud_lig = _combine_dock(de_lig, gd_sum, c_dock,
                            p["dock_e2"][0], row(p["dock_e2"][1]), 1)
    eud_int = _combine_dock(de_int, gd_sum, c_dock,
                            p["dock_e2"][0], row(p["dock_e2"][1]), 2)
    pd = _sc_scatter_dock(zed10, eud_rec, r_rec, eud_lig, r_lig4,
                          eud_int, r_int)

    nodes20 = jnp.concatenate([t10, nu_int], axis=0)
    q = _dock_final(
        nodes20, pd, p["dock_n1"][0], p["dock_n2"][0], row(p["dock_n2"][1]),
        cg_dock, g_dock, p["dock_g1"][0], row(p["dock_g1"][1]),
        p["dock_g2"][0], row(p["dock_g2"][1]),
        p["out"][0], row(p["out"][1]), p["value"][0], row(p["value"][1]))
    return q.reshape(1)


# speedup vs baseline: 3.6716x; 1.0550x over previous
import functools

import jax
import jax.numpy as jnp
from jax import lax
from jax.experimental import pallas as pl
from jax.experimental.pallas import tpu as pltpu
from jax.experimental.pallas import tpu_sc as plsc

C = 128
F32 = jnp.float32
NC, NS = 2, 16
NW = NC * NS
N5, N10, N20 = 5000, 10000, 20000
E1 = 160000
ED = 480000
NPAD5, NPAD10 = 5120, 10240


def _relu(x):
    return jnp.maximum(x, 0.0)


def _dot(a, b):
    return jnp.dot(a, b, preferred_element_type=F32)


def _prep_body(A2, a1w, a1b, a2w, a2b, encnw, encnb, encew, enceb,
               se1w, se1b, ie1w, ie1b, de1w, se2w, se2b, ie2w, ie2b,
               sn1w, sn1b, in1w, in1b,
               o_iden, o_act, o_encn3, o_bn3, o_ences, o_encei,
               o_crec, o_clig, o_cint, o_e2ds, o_b2ds, o_e2di, o_b2di,
               o_w10, o_cgrec, o_cglig, o_cgint):
    v = _dot(_relu(_dot(A2[...], a1w[...]) + a1b[...]), a2w[...]) + a2b[...]
    iden = v[0:1, :]
    act = v[1:2, :]
    o_iden[...] = iden
    o_act[...] = act
    Ws = se1w[C:2 * C, :]
    Wr = se1w[2 * C:3 * C, :]
    o_encn3[...] = jnp.concatenate(
        [encnw[...], _dot(encnw[...], Ws), _dot(encnw[...], Wr)], axis=1)
    o_bn3[...] = jnp.concatenate(
        [encnb[...], _dot(encnb[...], Ws), _dot(encnb[...], Wr)], axis=1)
    Wes = se1w[0:C, :]
    Wei = ie1w[0:C, :]
    o_ences[...] = _dot(encew[...], Wes)
    o_encei[...] = _dot(encew[...], Wei)
    bes = _dot(enceb[...], Wes)
    bei = _dot(enceb[...], Wei)
    o_crec[...] = bes + _dot(iden, se1w[3 * C:, :]) + se1b[...]
    o_clig[...] = bes + _dot(act, se1w[3 * C:, :]) + se1b[...]
    o_cint[...] = bei + _dot(act, ie1w[3 * C:, :]) + ie1b[...]
    Wd = de1w[0:C, :]
    o_e2ds[...] = jnp.concatenate([se2w[...], _dot(se2w[...], Wd)], axis=1)
    o_b2ds[...] = jnp.concatenate([se2b[...], _dot(se2b[...], Wd)], axis=1)
    o_e2di[...] = jnp.concatenate([ie2w[...], _dot(ie2w[...], Wd)], axis=1)
    o_b2di[...] = jnp.concatenate([ie2b[...], _dot(ie2b[...], Wd)], axis=1)
    o_w10[...] = jnp.concatenate(
        [ie1w[C:2 * C, :], ie1w[2 * C:3 * C, :],
         de1w[C:2 * C, :], de1w[2 * C:3 * C, :]], axis=1)
    o_cgrec[...] = _dot(iden, sn1w[2 * C:, :]) + sn1b[...]
    o_cglig[...] = _dot(act, sn1w[2 * C:, :]) + sn1b[...]
    o_cgint[...] = _dot(act, in1w[2 * C:, :]) + in1b[...]


def _prep(*args):
    s = lambda *sh: jax.ShapeDtypeStruct(sh, F32)
    outs = (s(1, C), s(1, C), s(C, 3 * C), s(1, 3 * C), s(16, C), s(16, C),
            s(1, C), s(1, C), s(1, C), s(C, 2 * C), s(1, 2 * C),
            s(C, 2 * C), s(1, 2 * C), s(C, 4 * C), s(1, C), s(1, C), s(1, C))
    return pl.pallas_call(_prep_body, out_shape=outs)(*args)


def _dock_consts_body(grec, glig, gint, de1w, de1b, dn1w, dn1b,
                      o_g, o_c, o_cg):
    g = grec[...] + glig[...] + gint[...]
    o_g[...] = g
    o_c[...] = _dot(g, de1w[3 * C:, :]) + de1b[...]
    o_cg[...] = _dot(g, dn1w[2 * C:, :]) + dn1b[...]


def _dock_consts(*args):
    s = lambda *sh: jax.ShapeDtypeStruct(sh, F32)
    return pl.pallas_call(_dock_consts_body,
                          out_shape=(s(1, C), s(1, C), s(1, C)))(*args)


def _mm_bias_body(x, w, b, o):
    o[...] = _dot(x[...], w[...]) + b[...]


def _encode_nodes(x10, w, b):
    return pl.pallas_call(
        _mm_bias_body,
        out_shape=jax.ShapeDtypeStruct((N10, 3 * C), F32))(x10, w, b)


def _mm_body(x, w, o):
    o[...] = _dot(x[...], w[...])


def _encode_edges(e_raw, w):
    E = e_raw.shape[0]
    T = 2000
    return pl.pallas_call(
        _mm_body,
        grid=(E // T,),
        in_specs=[pl.BlockSpec((T, 16), lambda i: (i, 0)),
                  pl.BlockSpec((16, C), lambda i: (0, 0))],
        out_specs=pl.BlockSpec((T, C), lambda i: (i, 0)),
        out_shape=jax.ShapeDtypeStruct((E, C), F32))(e_raw, w)


def _proj10(t10, w10):
    return pl.pallas_call(
        _mm_body,
        out_shape=jax.ShapeDtypeStruct((N10, 4 * C), F32))(t10, w10)


def _combine_body(ep, g, c, w, b, o_eu, o_de):
    h = _relu(ep[...] + g[...] + c[...])
    o = _dot(h, w[...]) + b[...]
    o_eu[...] = o[:, 0:C]
    o_de[...] = o[:, C:2 * C]


def _combine_edges(ep, g, c, w, b):
    E = ep.shape[0]
    T = 2000
    bs = lambda: pl.BlockSpec((T, C), lambda i: (i, 0))
    return pl.pallas_call(
        _combine_body,
        grid=(E // T,),
        in_specs=[bs(), bs(),
                  pl.BlockSpec((1, C), lambda i: (0, 0)),
                  pl.BlockSpec((C, 2 * C), lambda i: (0, 0)),
                  pl.BlockSpec((1, 2 * C), lambda i: (0, 0))],
        out_specs=(bs(), bs()),
        out_shape=(jax.ShapeDtypeStruct((E, C), F32),
                   jax.ShapeDtypeStruct((E, C), F32)))(ep, g, c, w, b)


def _combine_dock_body(de, g, c, w, b, o_eu):
    h = _relu(de[...] + g[...] + c[...])
    o_eu[...] = _dot(h, w[...]) + b[...]


def _combine_dock(de, g_full, c, w, b, seg):
    T = 2000
    off = seg * (E1 // T)
    bs = pl.BlockSpec((T, C), lambda i: (i, 0))
    bso = pl.BlockSpec((T, C), lambda i, o=off: (o + i, 0))
    return pl.pallas_call(
        _combine_dock_body,
        grid=(E1 // T,),
        in_specs=[bs, bso,
                  pl.BlockSpec((1, C), lambda i: (0, 0)),
                  pl.BlockSpec((C, C), lambda i: (0, 0)),
                  pl.BlockSpec((1, C), lambda i: (0, 0))],
        out_specs=bs,
        out_shape=jax.ShapeDtypeStruct((E1, C), F32))(
            de, g_full, c, w, b)


def _make_node_update(N, NPAD, E):
    def body(nodes, S, n1w, n2w, n2b, cg, g, g1w, g1b, g2w, g2b,
             o_nu, o_gu):
        agg = S[0:N, :] + S[NPAD:NPAD + N, :]
        hn = _relu(_dot(nodes[...], n1w[0:C, :]) + _dot(agg, n1w[C:2 * C, :])
                   + cg[...])
        nu = _dot(hn, n2w[...]) + n2b[...]
        o_nu[...] = nu
        mean_nu = jnp.sum(nu, axis=0, keepdims=True) * (1.0 / N)
        mean_eu = jnp.sum(agg, axis=0, keepdims=True) * (1.0 / E)
        gin = jnp.concatenate([mean_nu, mean_eu, g[...]], axis=1)
        o_gu[...] = _dot(_relu(_dot(gin, g1w[...]) + g1b[...]), g2w[...]) \
            + g2b[...]

    def call(*args):
        return pl.pallas_call(
            body,
            out_shape=(jax.ShapeDtypeStruct((N, C), F32),
                       jax.ShapeDtypeStruct((1, C), F32)))(*args)
    return call


_node_update5 = _make_node_update(N5, NPAD5, E1)
_node_update10 = _make_node_update(N10, NPAD10, E1)


def _dock_final_body(nodes, P, dn1w, dn2w, dn2b, cg, g, g1w, g1b, g2w, g2b,
                     outw, outb, valw, valb, o_q, acc):
    i = pl.program_id(0)

    @pl.when(i == 0)
    def _():
        acc[...] = jnp.zeros_like(acc)

    TN = 5000
    ii = jnp.minimum(i, 1)
    a0 = P[pl.ds(ii * TN, TN), :] + P[pl.ds(NPAD10 + ii * TN, TN), :]
    agg = a0 * jnp.where(i < 2, 1.0, 0.0)
    hn = _relu(_dot(nodes[...], dn1w[0:C, :]) + _dot(agg, dn1w[C:2 * C, :])
               + cg[...])
    nu = _dot(hn, dn2w[...]) + dn2b[...]
    acc[0:1, :] += jnp.sum(nu, axis=0, keepdims=True)
    acc[1:2, :] += jnp.sum(agg, axis=0, keepdims=True)

    @pl.when(i == 3)
    def _():
        mean_nu = acc[0:1, :] * (1.0 / N20)
        mean_eu = acc[1:2, :] * (1.0 / ED)
        gin = jnp.concatenate([mean_nu, mean_eu, g[...]], axis=1)
        gd = _dot(_relu(_dot(gin, g1w[...]) + g1b[...]), g2w[...]) + g2b[...]
        q = _relu(_dot(gd, outw[...]) + outb[...])
        q = jax.nn.sigmoid(_dot(q, valw[...]) + valb[...]) * 200.0 - 100.0
        o_q[...] = q


def _dock_final(nodes20, P, *weights):
    TN = 5000
    full = lambda r, c: pl.BlockSpec((r, c), lambda i: (0, 0))
    return pl.pallas_call(
        _dock_final_body,
        grid=(4,),
        in_specs=[pl.BlockSpec((TN, C), lambda i: (i, 0)),
                  full(2 * NPAD10, C),
                  full(3 * C, C), full(C, C), full(1, C), full(1, C),
                  full(1, C), full(3 * C, C), full(1, C), full(C, C),
                  full(1, C), full(C, C), full(1, C), full(C, 1),
                  full(1, 1)],
        out_specs=pl.BlockSpec((1, 1), lambda i: (0, 0)),
        out_shape=jax.ShapeDtypeStruct((1, 1), F32),
        scratch_shapes=[pltpu.VMEM((8, C), F32)])(nodes20, P, *weights)


def _make_sc_gather(E, K):
    EW = E // NW
    NFULL = EW // 128
    REM = EW - NFULL * 128
    NG = NFULL // K
    TAIL = NFULL - NG * K
    mesh = plsc.VectorSubcoreMesh(core_axis_name="c", subcore_axis_name="s",
                                  num_cores=NC, num_subcores=NS)
    scratch = [pltpu.VMEM((EW,), jnp.int32), pltpu.VMEM((EW,), jnp.int32),
               pltpu.VMEM((K, 128, C), F32), pltpu.VMEM((K, 128, C), F32),
               pltpu.SemaphoreType.DMA, pltpu.SemaphoreType.DMA]

    def kern(T1, T2, I1, I2, O1, iv1, iv2, rb1, rb2, sem_g, sem_w):
        wid = lax.axis_index("s") * NC + lax.axis_index("c")
        base = wid * EW
        pltpu.sync_copy(I1.at[pl.ds(base, EW)], iv1)
        pltpu.sync_copy(I2.at[pl.ds(base, EW)], iv2)

        def vsum(dst, src, nrow):
            def row(j, carry):
                for g in range(C // 16):
                    dst[j, pl.ds(g * 16, 16)] = (
                        dst[j, pl.ds(g * 16, 16)] + src[j, pl.ds(g * 16, 16)])
                return carry
            lax.fori_loop(0, nrow, row, 0)

        def group(goff, nk):
            gds = []
            for k in range(nk):
                loc = (goff + k) * 128
                gds.append(pltpu.async_copy(
                    T1.at[iv1.at[pl.ds(loc, 128)]], rb1.at[k], sem_g))
                gds.append(pltpu.async_copy(
                    T2.at[iv2.at[pl.ds(loc, 128)]], rb2.at[k], sem_g))
            wds = []
            for k in range(nk):
                gds[2 * k].wait()
                gds[2 * k + 1].wait()
                vsum(rb1.at[k], rb2.at[k], 128)
                glob = base + (goff + k) * 128
                wds.append(pltpu.async_copy(
                    rb1.at[k], O1.at[pl.ds(glob, 128)], sem_w))
            for d in wds:
                d.wait()

        def loop(g, carry):
            group(g * K, K)
            return carry

        lax.fori_loop(0, NG, loop, 0)
        if TAIL:
            group(NG * K, TAIL)
        if REM:
            loc = NFULL * 128
            d1 = pltpu.async_copy(
                T1.at[iv1.at[pl.ds(loc, REM)]],
                rb1.at[0, pl.ds(0, REM)], sem_g)
            d2 = pltpu.async_copy(
                T2.at[iv2.at[pl.ds(loc, REM)]],
                rb2.at[0, pl.ds(0, REM)], sem_g)
            d1.wait()
            d2.wait()
            vsum(rb1.at[0], rb2.at[0], REM)
            w1 = pltpu.async_copy(
                rb1.at[0, pl.ds(0, REM)], O1.at[pl.ds(base + loc, REM)],
                sem_w)
            w1.wait()

    out = jax.ShapeDtypeStruct((E, C), F32)
    return pl.kernel(kern, out_type=out, mesh=mesh, scratch_types=scratch)


_sc_gather_e1 = _make_sc_gather(E1, 3)
_sc_gather_ed = _make_sc_gather(ED, 2)


def _make_sc_scatter(nseg, NPAD, K):
    EW = E1 // NW
    NFULL = EW // 128
    REM = EW - NFULL * 128
    NGRP = NFULL // K
    TAILK = NFULL - NGRP * K
    RPS = NPAD // NS
    mesh = plsc.VectorSubcoreMesh(core_axis_name="c", subcore_axis_name="s",
                                  num_cores=NC, num_subcores=NS)
    scratch = ([pltpu.VMEM_SHARED((NPAD, C), F32),
                pltpu.VMEM((K, 128, C), F32)]
               + [pltpu.VMEM((128,), jnp.int32) for _ in range(K)]
               + [pltpu.VMEM((REM,), jnp.int32), pltpu.VMEM((REM, C), F32),
                  pltpu.SemaphoreType.DMA, pltpu.SemaphoreType.DMA])

    def kern(*refs):
        zed = refs[0]
        vi = refs[1:1 + 2 * nseg]
        out = refs[1 + 2 * nseg]
        rest = refs[2 + 2 * nseg:]
        S, vb = rest[0], rest[1]
        ibs = rest[2:2 + K]
        ibr, vbr, sem_v, sem_s = rest[2 + K:]
        cid = lax.axis_index("c")
        sid = lax.axis_index("s")
        wid = sid * NC + cid
        pltpu.sync_copy(zed.at[pl.ds(sid * RPS, RPS)],
                        S.at[pl.ds(sid * RPS, RPS)])
        plsc.subcore_barrier()
        base = wid * EW
        for seg in range(nseg):
            V, I = vi[2 * seg], vi[2 * seg + 1]

            def group(go, nk):
                lds = []
                for k in range(nk):
                    off = base + (go + k) * 128
                    lds.append(pltpu.async_copy(
                        I.at[pl.ds(off, 128)], ibs[k], sem_v))
                    lds.append(pltpu.async_copy(
                        V.at[pl.ds(off, 128)], vb.at[k], sem_v))
                sds = []
                for k in range(nk):
                    lds[2 * k].wait()
                    lds[2 * k + 1].wait()
                    sds.append(pltpu.async_copy(
                        vb.at[k], S.at[ibs[k]], sem_s, add=True))
                for d in sds:
                    d.wait()

            def loop(g, carry):
                group(g * K, K)
                return carry

            lax.fori_loop(0, NGRP, loop, 0)
            if TAILK:
                group(NGRP * K, TAILK)
            if REM:
                off = base + NFULL * 128
                pltpu.sync_copy(I.at[pl.ds(off, REM)], ibr)
                pltpu.sync_copy(V.at[pl.ds(off, REM)], vbr)
                pltpu.sync_copy(vbr, S.at[ibr], add=True)
        plsc.subcore_barrier()
        pltpu.sync_copy(S.at[pl.ds(sid * RPS, RPS)],
                        out.at[pl.ds(cid * NPAD + sid * RPS, RPS)])

    out = jax.ShapeDtypeStruct((2 * NPAD, C), F32)
    return pl.kernel(kern, out_type=out, mesh=mesh, scratch_types=scratch)


_sc_scatter5 = _make_sc_scatter(1, NPAD5, 4)
_sc_scatter10 = _make_sc_scatter(1, NPAD10, 2)
_sc_scatter_dock = _make_sc_scatter(3, NPAD10, 2)


def kernel(e_rec, s_rec, r_rec, n_rec, e_lig, s_lig, r_lig, n_lig,
           e_int, s_int, r_int, action, params):
    p = params
    row = lambda v: v.reshape(1, -1)
    A2 = jnp.zeros((8, 8), F32).at[0, 0].set(1.0).at[1].set(action)

    (iden, act, encn3, bn3, ences, encei, c_rec, c_lig, c_int,
     e2ds, b2ds, e2di, b2di, w10, cg_rec, cg_lig, cg_int) = _prep(
        A2, p["act1"][0], row(p["act1"][1]), p["act2"][0], row(p["act2"][1]),
        p["enc_n"][0], row(p["enc_n"][1]), p["enc_e"][0], row(p["enc_e"][1]),
        p["single_e1"][0], row(p["single_e1"][1]),
        p["inter_e1"][0], row(p["inter_e1"][1]),
        p["dock_e1"][0],
        p["single_e2"][0], row(p["single_e2"][1]),
        p["inter_e2"][0], row(p["inter_e2"][1]),
        p["single_n1"][0], row(p["single_n1"][1]),
        p["inter_n1"][0], row(p["inter_n1"][1]))

    x10 = jnp.concatenate([n_rec, n_lig], axis=0)
    y = _encode_nodes(x10, encn3, bn3)
    nr, ps_rec, pr_rec = y[:N5, :C], y[:N5, C:2 * C], y[:N5, 2 * C:]
    nl, ps_lig, pr_lig = y[N5:, :C], y[N5:, C:2 * C], y[N5:, 2 * C:]

    ep_rec = _encode_edges(e_rec, ences)
    ep_lig = _encode_edges(e_lig, ences)
    ep_int = _encode_edges(e_int, encei)

    zed5 = jnp.zeros((NPAD5, C), F32)
    zed10 = jnp.zeros((NPAD10, C), F32)

    gsr = _sc_gather_e1(ps_rec, pr_rec, s_rec, r_rec)
    eu_rec, de_rec = _combine_edges(ep_rec, gsr, c_rec, e2ds, b2ds)
    part = _sc_scatter5(zed5, eu_rec, r_rec)
    nu_rec, grec = _node_update5(
        nr, part, p["single_n1"][0], p["single_n2"][0], row(p["single_n2"][1]),
        cg_rec, iden, p["single_g1"][0], row(p["single_g1"][1]),
        p["single_g2"][0], row(p["single_g2"][1]))

    gsl = _sc_gather_e1(ps_lig, pr_lig, s_lig, r_lig)
    eu_lig, de_lig = _combine_edges(ep_lig, gsl, c_lig, e2ds, b2ds)
    part = _sc_scatter5(zed5, eu_lig, r_lig)
    nu_lig, glig = _node_update5(
        nl, part, p["single_n1"][0], p["single_n2"][0], row(p["single_n2"][1]),
        cg_lig, act, p["single_g1"][0], row(p["single_g1"][1]),
        p["single_g2"][0], row(p["single_g2"][1]))

    t10 = jnp.concatenate([nu_rec, nu_lig], axis=0)
    p10 = _proj10(t10, w10)
    gsi = _sc_gather_e1(p10[:, :C], p10[:, C:2 * C], s_int, r_int)
    eu_int, de_int = _combine_edges(ep_int, gsi, c_int, e2di, b2di)
    part = _sc_scatter10(zed10, eu_int, r_int)
    nu_int, gint = _node_update10(
        t10, part, p["inter_n1"][0], p["inter_n2"][0], row(p["inter_n2"][1]),
        cg_int, act, p["inter_g1"][0], row(p["inter_g1"][1]),
        p["inter_g2"][0], row(p["inter_g2"][1]))

    g_dock, c_dock, cg_dock = _dock_consts(
        grec, glig, gint, p["dock_e1"][0], row(p["dock_e1"][1]),
        p["dock_n1"][0], row(p["dock_n1"][1]))
    s_dock = jnp.concatenate([s_rec, s_lig + 400, s_int])
    r_lig4 = r_lig + 400
    r_dock = jnp.concatenate([r_rec, r_lig4, r_int])
    gd_sum = _sc_gather_ed(p10[:, 2 * C:3 * C], p10[:, 3 * C:],
                           s_dock, r_dock)
    eud_rec = _combine_dock(de_rec, gd_sum, c_dock,
                            p["dock_e2"][0], row(p["dock_e2"][1]), 0)
    eud_lig = _combine_dock(de_lig, gd_sum, c_dock,
                            p["dock_e2"][0], row(p["dock_e2"][1]), 1)
    eud_int = _combine_dock(de_int, gd_sum, c_dock,
                            p["dock_e2"][0], row(p["dock_e2"][1]), 2)
    pd = _sc_scatter_dock(zed10, eud_rec, r_rec, eud_lig, r_lig4,
                          eud_int, r_int)

    nodes20 = jnp.concatenate([t10, nu_int], axis=0)
    q = _dock_final(
        nodes20, pd, p["dock_n1"][0], p["dock_n2"][0], row(p["dock_n2"][1]),
        cg_dock, g_dock, p["dock_g1"][0], row(p["dock_g1"][1]),
        p["dock_g2"][0], row(p["dock_g2"][1]),
        p["out"][0], row(p["out"][1]), p["value"][0], row(p["value"][1]))
    return q.reshape(1)
